# baseline (speedup 1.0000x reference)
import functools

import jax
import jax.numpy as jnp
from jax.experimental import pallas as pl
from jax.experimental.pallas import tpu as pltpu

_NCORES = 2
_DEPTH = 6
_CN = 8192


def _stream_kernel(nchunks, x1_hbm, x2_hbm, w1_ref, w2_ref, b1_ref, b2_ref,
                   o_ref, bufh, bufl, semh, seml):
    base = pl.program_id(0) * (nchunks * _CN)
    D = x1_hbm.shape[0]
    lo = D - 8

    def _copies(slot, j):
        off = base + j * _CN
        return (
            pltpu.make_async_copy(
                x1_hbm.at[0:8, pl.ds(off, _CN)], bufh.at[slot, 0],
                semh.at[slot, 0]),
            pltpu.make_async_copy(
                x2_hbm.at[0:8, pl.ds(off, _CN)], bufh.at[slot, 1],
                semh.at[slot, 1]),
            pltpu.make_async_copy(
                x1_hbm.at[pl.ds(8, lo), pl.ds(off, _CN)], bufl.at[slot, 0],
                seml.at[slot, 0]),
            pltpu.make_async_copy(
                x2_hbm.at[pl.ds(8, lo), pl.ds(off, _CN)], bufl.at[slot, 1],
                seml.at[slot, 1]),
        )

    for jj in range(_DEPTH - 1):
        if jj < nchunks:
            for c in _copies(jj % _DEPTH, jj):
                c.start()

    w1 = w1_ref[...].reshape(D, 1)
    w2 = w2_ref[...].reshape(D, 1)
    w1h, w1l = w1[0:8, :], w1[8:D, :]
    w2h, w2l = w2[0:8, :], w2[8:D, :]
    bias = b1_ref[0] + b2_ref[0]

    def body(j, _):
        slot = jax.lax.rem(j, _DEPTH)
        jn = j + _DEPTH - 1

        @pl.when(jn < nchunks)
        def _():
            for c in _copies(jax.lax.rem(jn, _DEPTH), jn):
                c.start()

        for c in _copies(slot, j):
            c.wait()
        y = bufh[slot, 0] * w1h + bufh[slot, 1] * w2h
        z = bufl[slot, 0] * w1l + bufl[slot, 1] * w2l
        o_ref[pl.ds(j * _CN, _CN)] = (
            jnp.sum(y, axis=0) + jnp.sum(z, axis=0) + bias)
        return 0

    jax.lax.fori_loop(0, nchunks, body, 0, unroll=True)


@functools.partial(jax.jit, static_argnames=("nchunks",))
def _stream_call(x1t, x2t, w1r, w2r, b1r, b2r, nchunks):
    D, B = x1t.shape
    out = pl.pallas_call(
        functools.partial(_stream_kernel, nchunks),
        out_shape=jax.ShapeDtypeStruct((B,), jnp.float32),
        grid=(_NCORES,),
        in_specs=[
            pl.BlockSpec(memory_space=pltpu.MemorySpace.HBM),
            pl.BlockSpec(memory_space=pltpu.MemorySpace.HBM),
            pl.BlockSpec((1, D), lambda i: (0, 0)),
            pl.BlockSpec((1, D), lambda i: (0, 0)),
            pl.BlockSpec(memory_space=pltpu.MemorySpace.SMEM),
            pl.BlockSpec(memory_space=pltpu.MemorySpace.SMEM),
        ],
        out_specs=pl.BlockSpec((B // _NCORES,), lambda i: (i,)),
        scratch_shapes=[
            pltpu.VMEM((_DEPTH, 2, 8, _CN), jnp.float32),
            pltpu.VMEM((_DEPTH, 2, D - 8, _CN), jnp.float32),
            pltpu.SemaphoreType.DMA((_DEPTH, 2)),
            pltpu.SemaphoreType.DMA((_DEPTH, 2)),
        ],
        compiler_params=pltpu.CompilerParams(
            dimension_semantics=("parallel",),
        ),
    )(x1t, x2t, w1r, w2r, b1r, b2r)
    return out.reshape(B, 1)


def _colwise_kernel(x1_ref, x2_ref, w1_ref, w2_ref, b_ref, o_ref):
    y = x1_ref[...] * w1_ref[...] + x2_ref[...] * w2_ref[...]
    o_ref[...] = jnp.sum(y, axis=0) + b_ref[0]


@functools.partial(jax.jit, static_argnames=("bn",))
def _colwise_call(x1t, x2t, w1c, w2c, b, bn):
    D, B = x1t.shape
    out = pl.pallas_call(
        _colwise_kernel,
        out_shape=jax.ShapeDtypeStruct((B,), jnp.float32),
        grid=(pl.cdiv(B, bn),),
        in_specs=[
            pl.BlockSpec((D, bn), lambda i: (0, i)),
            pl.BlockSpec((D, bn), lambda i: (0, i)),
            pl.BlockSpec((D, 1), lambda i: (0, 0)),
            pl.BlockSpec((D, 1), lambda i: (0, 0)),
            pl.BlockSpec(memory_space=pltpu.MemorySpace.SMEM),
        ],
        out_specs=pl.BlockSpec((bn,), lambda i: (i,)),
        compiler_params=pltpu.CompilerParams(
            dimension_semantics=("parallel",),
        ),
    )(x1t, x2t, w1c, w2c, b)
    return out.reshape(B, 1)


def kernel(x1, x2, w1, b1, w2, b2):
    B, D = x1.shape
    if B % (_NCORES * _CN) == 0 and 8 < D <= 16:
        return _stream_call(
            x1.T, x2.T,
            w1.reshape(1, D).astype(jnp.float32),
            w2.reshape(1, D).astype(jnp.float32),
            jnp.ravel(b1).astype(jnp.float32),
            jnp.ravel(b2).astype(jnp.float32),
            B // (_NCORES * _CN))
    b = (jnp.ravel(b1) + jnp.ravel(b2)).astype(jnp.float32)
    return _colwise_call(
        x1.T, x2.T,
        w1.reshape(D, 1).astype(jnp.float32),
        w2.reshape(D, 1).astype(jnp.float32),
        b, min(32768, B))


# speedup vs baseline: 1.0337x; 1.0337x over previous
import functools

import jax
import jax.numpy as jnp
from jax.experimental import pallas as pl
from jax.experimental.pallas import tpu as pltpu

_NCORES = 2
_DEPTH = 8
_CN = 8192


def _stream_kernel(nchunks, x1_hbm, x2_hbm, w1_ref, w2_ref, b1_ref, b2_ref,
                   o_ref, bufh, bufl, semh, seml):
    base = pl.program_id(0) * (nchunks * _CN)
    D = x1_hbm.shape[0]
    lo = D - 8

    def _copies(slot, j):
        off = base + j * _CN
        return (
            pltpu.make_async_copy(
                x1_hbm.at[0:8, pl.ds(off, _CN)], bufh.at[slot, 0],
                semh.at[slot, 0]),
            pltpu.make_async_copy(
                x2_hbm.at[0:8, pl.ds(off, _CN)], bufh.at[slot, 1],
                semh.at[slot, 1]),
            pltpu.make_async_copy(
                x1_hbm.at[pl.ds(8, lo), pl.ds(off, _CN)], bufl.at[slot, 0],
                seml.at[slot, 0]),
            pltpu.make_async_copy(
                x2_hbm.at[pl.ds(8, lo), pl.ds(off, _CN)], bufl.at[slot, 1],
                seml.at[slot, 1]),
        )

    for jj in range(_DEPTH - 1):
        if jj < nchunks:
            for c in _copies(jj % _DEPTH, jj):
                c.start()

    w1 = w1_ref[...].reshape(D, 1)
    w2 = w2_ref[...].reshape(D, 1)
    w1h, w1l = w1[0:8, :], w1[8:D, :]
    w2h, w2l = w2[0:8, :], w2[8:D, :]
    bias = b1_ref[0] + b2_ref[0]

    def body(j, _):
        slot = jax.lax.rem(j, _DEPTH)
        jn = j + _DEPTH - 1

        @pl.when(jn < nchunks)
        def _():
            for c in _copies(jax.lax.rem(jn, _DEPTH), jn):
                c.start()

        for c in _copies(slot, j):
            c.wait()
        y = bufh[slot, 0] * w1h + bufh[slot, 1] * w2h
        z = bufl[slot, 0] * w1l + bufl[slot, 1] * w2l
        o_ref[pl.ds(j * _CN, _CN)] = (
            jnp.sum(y, axis=0) + jnp.sum(z, axis=0) + bias)
        return 0

    jax.lax.fori_loop(0, nchunks, body, 0, unroll=True)


@functools.partial(jax.jit, static_argnames=("nchunks",))
def _stream_call(x1t, x2t, w1r, w2r, b1r, b2r, nchunks):
    D, B = x1t.shape
    out = pl.pallas_call(
        functools.partial(_stream_kernel, nchunks),
        out_shape=jax.ShapeDtypeStruct((B,), jnp.float32),
        grid=(_NCORES,),
        in_specs=[
            pl.BlockSpec(memory_space=pltpu.MemorySpace.HBM),
            pl.BlockSpec(memory_space=pltpu.MemorySpace.HBM),
            pl.BlockSpec((1, D), lambda i: (0, 0)),
            pl.BlockSpec((1, D), lambda i: (0, 0)),
            pl.BlockSpec(memory_space=pltpu.MemorySpace.SMEM),
            pl.BlockSpec(memory_space=pltpu.MemorySpace.SMEM),
        ],
        out_specs=pl.BlockSpec((B // _NCORES,), lambda i: (i,)),
        scratch_shapes=[
            pltpu.VMEM((_DEPTH, 2, 8, _CN), jnp.float32),
            pltpu.VMEM((_DEPTH, 2, D - 8, _CN), jnp.float32),
            pltpu.SemaphoreType.DMA((_DEPTH, 2)),
            pltpu.SemaphoreType.DMA((_DEPTH, 2)),
        ],
        compiler_params=pltpu.CompilerParams(
            dimension_semantics=("parallel",),
        ),
    )(x1t, x2t, w1r, w2r, b1r, b2r)
    return out.reshape(B, 1)


def _colwise_kernel(x1_ref, x2_ref, w1_ref, w2_ref, b_ref, o_ref):
    y = x1_ref[...] * w1_ref[...] + x2_ref[...] * w2_ref[...]
    o_ref[...] = jnp.sum(y, axis=0) + b_ref[0]


@functools.partial(jax.jit, static_argnames=("bn",))
def _colwise_call(x1t, x2t, w1c, w2c, b, bn):
    D, B = x1t.shape
    out = pl.pallas_call(
        _colwise_kernel,
        out_shape=jax.ShapeDtypeStruct((B,), jnp.float32),
        grid=(pl.cdiv(B, bn),),
        in_specs=[
            pl.BlockSpec((D, bn), lambda i: (0, i)),
            pl.BlockSpec((D, bn), lambda i: (0, i)),
            pl.BlockSpec((D, 1), lambda i: (0, 0)),
            pl.BlockSpec((D, 1), lambda i: (0, 0)),
            pl.BlockSpec(memory_space=pltpu.MemorySpace.SMEM),
        ],
        out_specs=pl.BlockSpec((bn,), lambda i: (i,)),
        compiler_params=pltpu.CompilerParams(
            dimension_semantics=("parallel",),
        ),
    )(x1t, x2t, w1c, w2c, b)
    return out.reshape(B, 1)


def kernel(x1, x2, w1, b1, w2, b2):
    B, D = x1.shape
    if B % (_NCORES * _CN) == 0 and 8 < D <= 16:
        return _stream_call(
            x1.T, x2.T,
            w1.reshape(1, D).astype(jnp.float32),
            w2.reshape(1, D).astype(jnp.float32),
            jnp.ravel(b1).astype(jnp.float32),
            jnp.ravel(b2).astype(jnp.float32),
            B // (_NCORES * _CN))
    b = (jnp.ravel(b1) + jnp.ravel(b2)).astype(jnp.float32)
    return _colwise_call(
        x1.T, x2.T,
        w1.reshape(D, 1).astype(jnp.float32),
        w2.reshape(D, 1).astype(jnp.float32),
        b, min(32768, B))
